# Initial kernel scaffold; baseline (speedup 1.0000x reference)
import jax
import jax.numpy as jnp
from jax.experimental import pallas as pl


def kernel(x, edge_index, edge_attr, y, epoch, n_epoch, phi_to_W1, phi_to_b1, phi_to_W2, phi_to_b2, phi_from_W1, phi_from_b1, phi_from_W2, phi_from_b2, loop_W1, loop_b1, loop_W2, loop_b2, psy_W1, psy_b1, psy_W2, psy_b2, dec_W1, dec_b1, dec_W2, dec_b2):
    raise NotImplementedError("write your pallas kernel here")


# speedup vs baseline: 1.3737x; 1.3737x over previous
import functools
import jax
import jax.numpy as jnp
from jax import lax
from jax.experimental import pallas as pl
from jax.experimental.pallas import tpu as pltpu
from jax.experimental.pallas import tpu_sc as plsc

GAMMA = 0.5
ALPHA = 0.1

NC = 2
NS = 16
LANES = 16
CHUNK = 80
ZROWS = 64


def _edge_kernel(tabA, tabB, gA, gB, eaT, wv, out, cnt_out, sw_out,
                 gA_v, gB_v, s_v, ea0_v, ea1_v, rA, rB, orow,
                 w0, w1, zbuf, cnt_c, sw_c, red_c, stg_c,
                 acc, cnt_acc, sw_acc, sem1, sem2):
    E2 = gA.shape[0] // 2
    epw = E2 // NS
    nchunk = epw // CHUNK
    NP = acc.shape[0]
    rpt = NP // NS
    cid = lax.axis_index("c")
    tid = lax.axis_index("s")
    cN = cid * (tabA.shape[0] // 2)

    pltpu.sync_copy(wv.at[pl.ds(cid * 256, 128)], w0)
    pltpu.sync_copy(wv.at[pl.ds(cid * 256 + 128, 128)], w1)

    zv = jnp.zeros((LANES,), jnp.float32)

    def _zb(i, _):
        for j in range(128 // LANES):
            zbuf[i, pl.ds(j * LANES, LANES)] = zv
        return 0
    lax.fori_loop(0, ZROWS, _zb, 0)

    def _zr(i, _):
        red_c[pl.ds(i * LANES, LANES)] = zv
        return 0
    lax.fori_loop(0, rpt // LANES, _zr, 0)
    for b in range(rpt // ZROWS):
        pltpu.sync_copy(zbuf, acc.at[pl.ds(tid * rpt + b * ZROWS, ZROWS)])
    pltpu.sync_copy(red_c, cnt_acc.at[pl.ds(tid * rpt, rpt)])
    pltpu.sync_copy(red_c, sw_acc.at[pl.ds(tid * rpt, rpt)])
    plsc.subcore_barrier()

    def _chunk(k, _):
        ebe = tid * epw + k * CHUNK
        eb = cid * E2 + ebe
        pltpu.sync_copy(gA.at[pl.ds(eb, CHUNK)], gA_v)
        pltpu.sync_copy(gB.at[pl.ds(eb, CHUNK)], gB_v)
        pltpu.sync_copy(eaT.at[pl.ds(ebe, CHUNK)], ea0_v)
        pltpu.sync_copy(eaT.at[pl.ds(E2 + ebe, CHUNK)], ea1_v)
        cpA = pltpu.async_copy(tabA.at[gA_v], rA, sem1)
        cpB = pltpu.async_copy(tabB.at[gB_v], rB, sem2)
        cpA.wait()
        cpB.wait()

        def _group(g, _):
            base = g * LANES
            a0v = ea0_v[pl.ds(base, LANES)]
            a1v = ea1_v[pl.ds(base, LANES)]
            gav = gA_v[pl.ds(base, LANES)]
            gbv = gB_v[pl.ds(base, LANES)]
            mv = jnp.where(gav == gbv, 0.0, 1.0)
            s_v[pl.ds(base, LANES)] = gav - cN
            cnt_c[pl.ds(base, LANES)] = mv
            sw_c[pl.ds(base, LANES)] = a0v * (1.0 - mv)
            for ii in range(LANES):
                i = base + ii
                a0 = a0v[ii]
                a1 = a1v[ii]
                m = mv[ii]
                for j in range(8):
                    sl = pl.ds(j * LANES, LANES)
                    v = rA[i, sl] + rB[i, sl] + a0 * w0[sl] + a1 * w1[sl]
                    orow[i, sl] = jnp.maximum(v, 0.0) * m
            return 0
        lax.fori_loop(0, CHUNK // LANES, _group, 0)
        pltpu.sync_copy(orow, acc.at[s_v], add=True)
        pltpu.sync_copy(cnt_c, cnt_acc.at[s_v], add=True)
        pltpu.sync_copy(sw_c, sw_acc.at[s_v], add=True)
        return 0
    lax.fori_loop(0, nchunk, _chunk, 0)
    plsc.subcore_barrier()

    pltpu.sync_copy(cnt_acc.at[pl.ds(tid * rpt, rpt)], stg_c)
    pltpu.sync_copy(stg_c, cnt_out.at[pl.ds(cid * NP + tid * rpt, rpt)])
    pltpu.sync_copy(sw_acc.at[pl.ds(tid * rpt, rpt)], stg_c)
    pltpu.sync_copy(stg_c, sw_out.at[pl.ds(cid * NP + tid * rpt, rpt)])
    for b in range(rpt // ZROWS):
        r = tid * rpt + b * ZROWS
        pltpu.sync_copy(acc.at[pl.ds(r, ZROWS)], zbuf)
        pltpu.sync_copy(zbuf, out.at[pl.ds(cid * NP + r, ZROWS)])


def _make_edge_call(N2, E, NP):
    mesh = plsc.VectorSubcoreMesh(core_axis_name="c", subcore_axis_name="s")
    rpt = NP // NS
    return functools.partial(
        pl.kernel, mesh=mesh,
        out_type=[
            jax.ShapeDtypeStruct((2 * NP, 128), jnp.float32),
            jax.ShapeDtypeStruct((2 * NP,), jnp.float32),
            jax.ShapeDtypeStruct((2 * NP,), jnp.float32),
        ],
        scratch_types=[
            pltpu.VMEM((CHUNK,), jnp.int32),
            pltpu.VMEM((CHUNK,), jnp.int32),
            pltpu.VMEM((CHUNK,), jnp.int32),
            pltpu.VMEM((CHUNK,), jnp.float32),
            pltpu.VMEM((CHUNK,), jnp.float32),
            pltpu.VMEM((CHUNK, 128), jnp.float32),
            pltpu.VMEM((CHUNK, 128), jnp.float32),
            pltpu.VMEM((CHUNK, 128), jnp.float32),
            pltpu.VMEM((128,), jnp.float32),
            pltpu.VMEM((128,), jnp.float32),
            pltpu.VMEM((ZROWS, 128), jnp.float32),
            pltpu.VMEM((CHUNK,), jnp.float32),
            pltpu.VMEM((CHUNK,), jnp.float32),
            pltpu.VMEM((rpt,), jnp.float32),
            pltpu.VMEM((rpt,), jnp.float32),
            pltpu.VMEM_SHARED((NP, 128), jnp.float32),
            pltpu.VMEM_SHARED((NP,), jnp.float32),
            pltpu.VMEM_SHARED((NP,), jnp.float32),
            pltpu.SemaphoreType.DMA,
            pltpu.SemaphoreType.DMA,
        ],
    )(_edge_kernel)


def _tables_body(H_ref, Wa_ref, ba_ref, Wb_ref, tabA_ref, tabB_ref):
    h = H_ref[...]
    for c in range(2):
        tabA_ref[c] = jnp.dot(h, Wa_ref[c], preferred_element_type=jnp.float32) + ba_ref[c]
        tabB_ref[c] = jnp.dot(h, Wb_ref[c], preferred_element_type=jnp.float32)


def _finalize_body(H_ref, S_ref, cl_ref, x_ref, y_ref,
                   W2to_ref, b2to_ref, W2fr_ref, b2fr_ref,
                   Wl_ref, wlc_ref, b1l_ref, W2l_ref, b2l_ref,
                   Wph_ref, Wpt_ref, Wpf_ref, Wpl_ref, Wpx_ref, b1p_ref,
                   W2p_ref, b2p_ref, Wd1_ref, bd1_ref, Wd2_ref, bd2_ref,
                   Hn_ref, F_ref, l6_ref, l2_ref):
    H = H_ref[...]
    dot = functools.partial(jnp.dot, preferred_element_type=jnp.float32)
    cnt_to = cl_ref[0, :, 0]
    cnt_fr = cl_ref[1, :, 0]
    lf = -cl_ref[0, :, 1]
    mess_to = (dot(S_ref[0], W2to_ref[...]) + cnt_to[:, None] * b2to_ref[...]) \
        / jnp.maximum(cnt_to, 1.0)[:, None]
    mess_fr = (dot(S_ref[1], W2fr_ref[...]) + cnt_fr[:, None] * b2fr_ref[...]) \
        / jnp.maximum(cnt_fr, 1.0)[:, None]
    loop = dot(jnp.maximum(dot(H, Wl_ref[...]) + lf[:, None] * wlc_ref[...]
                           + b1l_ref[...], 0.0), W2l_ref[...]) + b2l_ref[...]
    hid = jnp.maximum(dot(H, Wph_ref[...]) + dot(mess_to, Wpt_ref[...])
                      + dot(mess_fr, Wpf_ref[...]) + dot(loop, Wpl_ref[...])
                      + dot(x_ref[...], Wpx_ref[...]) + b1p_ref[...], 0.0)
    Hn = H + ALPHA * (dot(hid, W2p_ref[...]) + b2p_ref[...])
    F = dot(jnp.maximum(dot(Hn, Wd1_ref[...]) + bd1_ref[...], 0.0),
            Wd2_ref[...]) + bd2_ref[...]
    d = F - y_ref[...]
    d2 = d * d
    Hn_ref[...] = Hn
    F_ref[...] = F
    l6_ref[...] = jnp.full((1, 1, 128), jnp.sum(d2 * d2 * d2), jnp.float32)
    l2_ref[...] = jnp.full((1, 1, 128), jnp.sum(d2), jnp.float32)


def kernel(x, edge_index, edge_attr, y, epoch, n_epoch,
           phi_to_W1, phi_to_b1, phi_to_W2, phi_to_b2,
           phi_from_W1, phi_from_b1, phi_from_W2, phi_from_b2,
           loop_W1, loop_b1, loop_W2, loop_b2,
           psy_W1, psy_b1, psy_W2, psy_b2,
           dec_W1, dec_b1, dec_W2, dec_b2):
    N = x.shape[0]
    E = edge_index.shape[1]
    L = dec_W1.shape[1]
    K = dec_W1.shape[0]
    NP = ((N + NS * ZROWS - 1) // (NS * ZROWS)) * (NS * ZROWS)
    BN = 2000
    grid = N // BN

    src = edge_index[0]
    dst = edge_index[1]
    gA = jnp.stack([dst, src + N]).astype(jnp.int32).ravel()
    gB = jnp.stack([src, dst + N]).astype(jnp.int32).ravel()
    eaT = edge_attr.T.ravel()
    edge_call = _make_edge_call(2 * N, E, NP)

    tables_call = pl.pallas_call(
        _tables_body,
        grid=(grid,),
        in_specs=[
            pl.BlockSpec((BN, L), lambda b: (b, 0)),
            pl.BlockSpec((2, L, L), lambda b: (0, 0, 0)),
            pl.BlockSpec((2, 1, L), lambda b: (0, 0, 0)),
            pl.BlockSpec((2, L, L), lambda b: (0, 0, 0)),
        ],
        out_specs=[
            pl.BlockSpec((2, BN, L), lambda b: (0, b, 0)),
            pl.BlockSpec((2, BN, L), lambda b: (0, b, 0)),
        ],
        out_shape=[
            jax.ShapeDtypeStruct((2, N, L), jnp.float32),
            jax.ShapeDtypeStruct((2, N, L), jnp.float32),
        ],
    )

    wspec = pl.BlockSpec((L, L), lambda b: (0, 0))
    bspec = pl.BlockSpec((1, L), lambda b: (0, 0))
    fin_call = pl.pallas_call(
        _finalize_body,
        grid=(grid,),
        in_specs=[
            pl.BlockSpec((BN, L), lambda b: (b, 0)),
            pl.BlockSpec((2, BN, L), lambda b: (0, b, 0)),
            pl.BlockSpec((2, BN, 2), lambda b: (0, b, 0)),
            pl.BlockSpec((BN, 8), lambda b: (b, 0)),
            pl.BlockSpec((BN, 8), lambda b: (b, 0)),
            wspec, bspec, wspec, bspec,
            wspec, bspec, bspec, wspec, bspec,
            wspec, wspec, wspec, wspec,
            pl.BlockSpec((8, L), lambda b: (0, 0)), bspec,
            wspec, bspec, wspec, bspec,
            pl.BlockSpec((L, 8), lambda b: (0, 0)),
            pl.BlockSpec((1, 8), lambda b: (0, 0)),
        ],
        out_specs=[
            pl.BlockSpec((BN, L), lambda b: (b, 0)),
            pl.BlockSpec((BN, 8), lambda b: (b, 0)),
            pl.BlockSpec((1, 1, 128), lambda b: (b, 0, 0)),
            pl.BlockSpec((1, 1, 128), lambda b: (b, 0, 0)),
        ],
        out_shape=[
            jax.ShapeDtypeStruct((N, L), jnp.float32),
            jax.ShapeDtypeStruct((N, 8), jnp.float32),
            jax.ShapeDtypeStruct((grid, 1, 128), jnp.float32),
            jax.ShapeDtypeStruct((grid, 1, 128), jnp.float32),
        ],
    )

    x8 = jnp.pad(x, ((0, 0), (0, 8 - x.shape[1])))
    y8 = jnp.pad(y, ((0, 0), (0, 8 - y.shape[1])))

    H = jnp.zeros((N, L), jnp.float32)
    tl1 = jnp.float32(0.0)
    tl2 = jnp.float32(0.0)
    F = None
    for u in range(K):
        Wa = jnp.stack([phi_to_W1[u, :L], phi_from_W1[u, :L]])
        ba = jnp.stack([phi_to_b1[u], phi_from_b1[u]])[:, None, :]
        Wb = jnp.stack([phi_to_W1[u, L:2 * L], phi_from_W1[u, L:2 * L]])
        wv = jnp.stack([phi_to_W1[u, 2 * L:], phi_from_W1[u, 2 * L:]]).ravel()
        tabA, tabB = tables_call(H, Wa, ba, Wb)
        acc, cnt_o, sw_o = edge_call(tabA.reshape(2 * N, L),
                                     tabB.reshape(2 * N, L), gA, gB, eaT, wv)
        S2 = acc.reshape(2, NP, L)[:, :N, :]
        cl = jnp.stack([cnt_o.reshape(2, NP)[:, :N],
                        sw_o.reshape(2, NP)[:, :N]], axis=-1)
        Wd2p = jnp.pad(dec_W2[u], ((0, 0), (0, 6)))
        bd2p = jnp.pad(dec_b2[u], ((0, 6)))[None, :]
        Wpxp = jnp.pad(psy_W1[u, 4 * L:], ((0, 5), (0, 0)))
        H, F8, l6, l2 = fin_call(
            H, S2, cl, x8, y8,
            phi_to_W2[u], phi_to_b2[u][None, :],
            phi_from_W2[u], phi_from_b2[u][None, :],
            loop_W1[u, :L] + loop_W1[u, L:2 * L], loop_W1[u, 2 * L][None, :],
            loop_b1[u][None, :], loop_W2[u], loop_b2[u][None, :],
            psy_W1[u, :L], psy_W1[u, L:2 * L], psy_W1[u, 2 * L:3 * L],
            psy_W1[u, 3 * L:4 * L], Wpxp, psy_b1[u][None, :],
            psy_W2[u], psy_b2[u][None, :],
            dec_W1[u], dec_b1[u][None, :], Wd2p, bd2p)
        F = F8[:, :2]
        w = GAMMA ** (K - u - 1)
        tl1 = tl1 + jnp.sum(l6[:, 0, 0]) ** (1.0 / 6.0) * w
        tl2 = tl2 + jnp.sqrt(jnp.sum(l2[:, 0, 0])) * w
    return (F, tl1, tl2)


# speedup vs baseline: 2.2580x; 1.6437x over previous
import functools
import jax
import jax.numpy as jnp
from jax import lax
from jax.experimental import pallas as pl
from jax.experimental.pallas import tpu as pltpu
from jax.experimental.pallas import tpu_sc as plsc

GAMMA = 0.5
ALPHA = 0.1

NC = 2
NS = 16
LANES = 16
CHUNK = 80
ZROWS = 32


def _edge_kernel(tabA, tabB, gAB, eaC, wv, out, cnt_out, sw_out,
                 ab_v, ea_v, s_v, rA, rB, orow,
                 w0, w1, zbuf, cnt_c, sw_c, stg_c,
                 acc, cnt_acc, sw_acc, semA, semB):
    E2 = gAB.shape[0] // 4
    epw = E2 // NS
    nchunk = epw // CHUNK
    NP = acc.shape[0]
    rpt = NP // NS
    cid = lax.axis_index("c")
    tid = lax.axis_index("s")
    cN = cid * (tabA.shape[0] // 2)

    pltpu.sync_copy(wv.at[pl.ds(cid * 256, 128)], w0)
    pltpu.sync_copy(wv.at[pl.ds(cid * 256 + 128, 128)], w1)
    w0v = [w0[pl.ds(j * LANES, LANES)] for j in range(8)]
    w1v = [w1[pl.ds(j * LANES, LANES)] for j in range(8)]

    zv = jnp.zeros((LANES,), jnp.float32)

    def _zb(i, _):
        for j in range(128 // LANES):
            zbuf[i, pl.ds(j * LANES, LANES)] = zv
        return 0
    lax.fori_loop(0, ZROWS, _zb, 0)

    def _zr(i, _):
        stg_c[pl.ds(i * LANES, LANES)] = zv
        return 0
    lax.fori_loop(0, rpt // LANES, _zr, 0)
    for b in range(rpt // ZROWS):
        pltpu.sync_copy(zbuf, acc.at[pl.ds(tid * rpt + b * ZROWS, ZROWS)])
    pltpu.sync_copy(stg_c, cnt_acc.at[pl.ds(tid * rpt, rpt)])
    pltpu.sync_copy(stg_c, sw_acc.at[pl.ds(tid * rpt, rpt)])
    plsc.subcore_barrier()

    def _chunk(k, _):
        gc = (cid * E2 + tid * epw) // CHUNK + k
        ge = (tid * epw) // CHUNK + k
        pltpu.sync_copy(gAB.at[pl.ds(gc * 2 * CHUNK, 2 * CHUNK)], ab_v)
        pltpu.sync_copy(eaC.at[pl.ds(ge * 2 * CHUNK, 2 * CHUNK)], ea_v)
        cpA = pltpu.async_copy(tabA.at[ab_v.at[pl.ds(0, CHUNK)]], rA, semA)
        cpB = pltpu.async_copy(tabB.at[ab_v.at[pl.ds(CHUNK, CHUNK)]], rB,
                               semB)
        cpA.wait()
        cpB.wait()

        def _group(g, _):
            base = g * LANES
            a0v = plsc.bitcast(ea_v[pl.ds(base, LANES)], jnp.float32)
            a1v = plsc.bitcast(ea_v[pl.ds(CHUNK + base, LANES)], jnp.float32)
            gav = ab_v[pl.ds(base, LANES)]
            gbv = ab_v[pl.ds(CHUNK + base, LANES)]
            mv = jnp.where(gav == gbv, 0.0, 1.0)
            s_v[pl.ds(base, LANES)] = gav - cN
            cnt_c[pl.ds(base, LANES)] = mv
            sw_c[pl.ds(base, LANES)] = a0v * (1.0 - mv)
            for ii in range(LANES):
                i = base + ii
                a0 = a0v[ii]
                a1 = a1v[ii]
                m = mv[ii]
                for j in range(8):
                    sl = pl.ds(j * LANES, LANES)
                    v = a0 * w0v[j] + a1 * w1v[j] + rA[i, sl] + rB[i, sl]
                    orow[i, sl] = jnp.maximum(v, 0.0) * m
            return 0
        lax.fori_loop(0, CHUNK // LANES, _group, 0)
        pltpu.sync_copy(orow, acc.at[s_v], add=True)
        pltpu.sync_copy(cnt_c, cnt_acc.at[s_v], add=True)
        pltpu.sync_copy(sw_c, sw_acc.at[s_v], add=True)
        return 0
    lax.fori_loop(0, nchunk, _chunk, 0)
    plsc.subcore_barrier()

    pltpu.sync_copy(cnt_acc.at[pl.ds(tid * rpt, rpt)], stg_c)
    pltpu.sync_copy(stg_c, cnt_out.at[pl.ds(cid * NP + tid * rpt, rpt)])
    pltpu.sync_copy(sw_acc.at[pl.ds(tid * rpt, rpt)], stg_c)
    pltpu.sync_copy(stg_c, sw_out.at[pl.ds(cid * NP + tid * rpt, rpt)])
    for b in range(rpt // ZROWS):
        r = tid * rpt + b * ZROWS
        pltpu.sync_copy(acc.at[pl.ds(r, ZROWS)], zbuf)
        pltpu.sync_copy(zbuf, out.at[pl.ds(cid * NP + r, ZROWS)])


def _make_edge_call(NP):
    mesh = plsc.VectorSubcoreMesh(core_axis_name="c", subcore_axis_name="s")
    rpt = NP // NS
    scratch = (
        [pltpu.VMEM((2 * CHUNK,), jnp.int32)] * 2
        + [pltpu.VMEM((CHUNK,), jnp.int32)]
        + [pltpu.VMEM((CHUNK, 128), jnp.float32)] * 3
        + [pltpu.VMEM((128,), jnp.float32)] * 2
        + [pltpu.VMEM((ZROWS, 128), jnp.float32)]
        + [pltpu.VMEM((CHUNK,), jnp.float32)] * 2
        + [pltpu.VMEM((rpt,), jnp.float32)]
        + [pltpu.VMEM_SHARED((NP, 128), jnp.float32)]
        + [pltpu.VMEM_SHARED((NP,), jnp.float32)] * 2
        + [pltpu.SemaphoreType.DMA] * 2
    )
    return functools.partial(
        pl.kernel, mesh=mesh,
        out_type=[
            jax.ShapeDtypeStruct((2 * NP, 128), jnp.float32),
            jax.ShapeDtypeStruct((2 * NP,), jnp.float32),
            jax.ShapeDtypeStruct((2 * NP,), jnp.float32),
        ],
        scratch_types=scratch,
        compiler_params=pltpu.CompilerParams(needs_layout_passes=False),
    )(_edge_kernel)


def _tables_body(H_ref, Wa_ref, ba_ref, Wb_ref, tabA_ref, tabB_ref):
    h = H_ref[...]
    for c in range(2):
        tabA_ref[c] = jnp.dot(h, Wa_ref[c],
                              preferred_element_type=jnp.float32) + ba_ref[c]
        tabB_ref[c] = jnp.dot(h, Wb_ref[c],
                              preferred_element_type=jnp.float32)


def _finalize_body(H_ref, S_ref, cl_ref, x_ref, y_ref,
                   W2to_ref, b2to_ref, W2fr_ref, b2fr_ref,
                   Wl_ref, wlc_ref, b1l_ref, W2l_ref, b2l_ref,
                   Wph_ref, Wpt_ref, Wpf_ref, Wpl_ref, Wpx_ref, b1p_ref,
                   W2p_ref, b2p_ref, Wd1_ref, bd1_ref, Wd2_ref, bd2_ref,
                   Hn_ref, F_ref, l6_ref, l2_ref):
    H = H_ref[...]
    dot = functools.partial(jnp.dot, preferred_element_type=jnp.float32)
    cnt_to = cl_ref[0, :, 0]
    cnt_fr = cl_ref[1, :, 0]
    lf = -cl_ref[0, :, 1]
    mess_to = (dot(S_ref[0], W2to_ref[...]) + cnt_to[:, None] * b2to_ref[...]) \
        / jnp.maximum(cnt_to, 1.0)[:, None]
    mess_fr = (dot(S_ref[1], W2fr_ref[...]) + cnt_fr[:, None] * b2fr_ref[...]) \
        / jnp.maximum(cnt_fr, 1.0)[:, None]
    loop = dot(jnp.maximum(dot(H, Wl_ref[...]) + lf[:, None] * wlc_ref[...]
                           + b1l_ref[...], 0.0), W2l_ref[...]) + b2l_ref[...]
    hid = jnp.maximum(dot(H, Wph_ref[...]) + dot(mess_to, Wpt_ref[...])
                      + dot(mess_fr, Wpf_ref[...]) + dot(loop, Wpl_ref[...])
                      + dot(x_ref[...], Wpx_ref[...]) + b1p_ref[...], 0.0)
    Hn = H + ALPHA * (dot(hid, W2p_ref[...]) + b2p_ref[...])
    F = dot(jnp.maximum(dot(Hn, Wd1_ref[...]) + bd1_ref[...], 0.0),
            Wd2_ref[...]) + bd2_ref[...]
    d = F - y_ref[...]
    d2 = d * d
    Hn_ref[...] = Hn
    F_ref[...] = F
    l6_ref[...] = jnp.full((1, 1, 128), jnp.sum(d2 * d2 * d2), jnp.float32)
    l2_ref[...] = jnp.full((1, 1, 128), jnp.sum(d2), jnp.float32)


def kernel(x, edge_index, edge_attr, y, epoch, n_epoch,
           phi_to_W1, phi_to_b1, phi_to_W2, phi_to_b2,
           phi_from_W1, phi_from_b1, phi_from_W2, phi_from_b2,
           loop_W1, loop_b1, loop_W2, loop_b2,
           psy_W1, psy_b1, psy_W2, psy_b2,
           dec_W1, dec_b1, dec_W2, dec_b2):
    N = x.shape[0]
    E = edge_index.shape[1]
    L = dec_W1.shape[1]
    K = dec_W1.shape[0]
    NP = ((N + NS * ZROWS - 1) // (NS * ZROWS)) * (NS * ZROWS)
    BN = 2000
    grid = N // BN

    src = edge_index[0]
    dst = edge_index[1]
    gA = jnp.stack([dst, src + N]).astype(jnp.int32).reshape(-1, CHUNK)
    gB = jnp.stack([src, dst + N]).astype(jnp.int32).reshape(-1, CHUNK)
    gAB = jnp.concatenate([gA, gB], axis=1).ravel()
    eaI = jax.lax.bitcast_convert_type(edge_attr.T, jnp.int32).reshape(-1, CHUNK)
    eaC = jnp.concatenate([eaI[:E // CHUNK], eaI[E // CHUNK:]], axis=1).ravel()
    edge_call = _make_edge_call(NP)

    tables_call = pl.pallas_call(
        _tables_body,
        grid=(grid,),
        in_specs=[
            pl.BlockSpec((BN, L), lambda b: (b, 0)),
            pl.BlockSpec((2, L, L), lambda b: (0, 0, 0)),
            pl.BlockSpec((2, 1, L), lambda b: (0, 0, 0)),
            pl.BlockSpec((2, L, L), lambda b: (0, 0, 0)),
        ],
        out_specs=[
            pl.BlockSpec((2, BN, L), lambda b: (0, b, 0)),
            pl.BlockSpec((2, BN, L), lambda b: (0, b, 0)),
        ],
        out_shape=[
            jax.ShapeDtypeStruct((2, N, L), jnp.float32),
            jax.ShapeDtypeStruct((2, N, L), jnp.float32),
        ],
    )

    wspec = pl.BlockSpec((L, L), lambda b: (0, 0))
    bspec = pl.BlockSpec((1, L), lambda b: (0, 0))
    fin_call = pl.pallas_call(
        _finalize_body,
        grid=(grid,),
        in_specs=[
            pl.BlockSpec((BN, L), lambda b: (b, 0)),
            pl.BlockSpec((2, BN, L), lambda b: (0, b, 0)),
            pl.BlockSpec((2, BN, 2), lambda b: (0, b, 0)),
            pl.BlockSpec((BN, 8), lambda b: (b, 0)),
            pl.BlockSpec((BN, 8), lambda b: (b, 0)),
            wspec, bspec, wspec, bspec,
            wspec, bspec, bspec, wspec, bspec,
            wspec, wspec, wspec, wspec,
            pl.BlockSpec((8, L), lambda b: (0, 0)), bspec,
            wspec, bspec, wspec, bspec,
            pl.BlockSpec((L, 8), lambda b: (0, 0)),
            pl.BlockSpec((1, 8), lambda b: (0, 0)),
        ],
        out_specs=[
            pl.BlockSpec((BN, L), lambda b: (b, 0)),
            pl.BlockSpec((BN, 8), lambda b: (b, 0)),
            pl.BlockSpec((1, 1, 128), lambda b: (b, 0, 0)),
            pl.BlockSpec((1, 1, 128), lambda b: (b, 0, 0)),
        ],
        out_shape=[
            jax.ShapeDtypeStruct((N, L), jnp.float32),
            jax.ShapeDtypeStruct((N, 8), jnp.float32),
            jax.ShapeDtypeStruct((grid, 1, 128), jnp.float32),
            jax.ShapeDtypeStruct((grid, 1, 128), jnp.float32),
        ],
    )

    x8 = jnp.pad(x, ((0, 0), (0, 8 - x.shape[1])))
    y8 = jnp.pad(y, ((0, 0), (0, 8 - y.shape[1])))

    H = jnp.zeros((N, L), jnp.float32)
    tl1 = jnp.float32(0.0)
    tl2 = jnp.float32(0.0)
    F = None
    for u in range(K):
        Wa = jnp.stack([phi_to_W1[u, :L], phi_from_W1[u, :L]])
        ba = jnp.stack([phi_to_b1[u], phi_from_b1[u]])[:, None, :]
        Wb = jnp.stack([phi_to_W1[u, L:2 * L], phi_from_W1[u, L:2 * L]])
        wv = jnp.stack([phi_to_W1[u, 2 * L:],
                        phi_from_W1[u, 2 * L:]]).ravel()
        tabA, tabB = tables_call(H, Wa, ba, Wb)
        acc, cnt_o, sw_o = edge_call(tabA.reshape(2 * N, L),
                                     tabB.reshape(2 * N, L), gAB, eaC, wv)
        S2 = acc.reshape(2, NP, L)[:, :N, :]
        cl = jnp.stack([cnt_o.reshape(2, NP)[:, :N],
                        sw_o.reshape(2, NP)[:, :N]], axis=-1)
        Wd2p = jnp.pad(dec_W2[u], ((0, 0), (0, 6)))
        bd2p = jnp.pad(dec_b2[u], ((0, 6)))[None, :]
        Wpxp = jnp.pad(psy_W1[u, 4 * L:], ((0, 5), (0, 0)))
        H, F8, l6, l2 = fin_call(
            H, S2, cl, x8, y8,
            phi_to_W2[u], phi_to_b2[u][None, :],
            phi_from_W2[u], phi_from_b2[u][None, :],
            loop_W1[u, :L] + loop_W1[u, L:2 * L], loop_W1[u, 2 * L][None, :],
            loop_b1[u][None, :], loop_W2[u], loop_b2[u][None, :],
            psy_W1[u, :L], psy_W1[u, L:2 * L], psy_W1[u, 2 * L:3 * L],
            psy_W1[u, 3 * L:4 * L], Wpxp, psy_b1[u][None, :],
            psy_W2[u], psy_b2[u][None, :],
            dec_W1[u], dec_b1[u][None, :], Wd2p, bd2p)
        F = F8[:, :2]
        w = GAMMA ** (K - u - 1)
        tl1 = tl1 + jnp.sum(l6[:, 0, 0]) ** (1.0 / 6.0) * w
        tl2 = tl2 + jnp.sqrt(jnp.sum(l2[:, 0, 0])) * w
    return (F, tl1, tl2)


# speedup vs baseline: 2.3808x; 1.0544x over previous
import functools
import jax
import jax.numpy as jnp
from jax import lax
from jax.experimental import pallas as pl
from jax.experimental.pallas import tpu as pltpu
from jax.experimental.pallas import tpu_sc as plsc

GAMMA = 0.5
ALPHA = 0.1

NC = 2
NS = 16
LANES = 16
CHUNK = 80
ZROWS = 16


def _edge_kernel(tabA, tabB, gAB, eaC, wv, out, cnt_out, sw_out,
                 ab_v, ea_v, s_v0, s_v1, rA, rB, orow0, orow1,
                 w0, w1, zbuf, cnt_c0, cnt_c1, sw_c0, sw_c1, stg_c,
                 acc, cnt_acc, sw_acc, semA, semB, semS):
    E2 = gAB.shape[0] // 4
    epw = E2 // NS
    nchunk = epw // CHUNK
    NP = acc.shape[0]
    rpt = NP // NS
    cid = lax.axis_index("c")
    tid = lax.axis_index("s")
    cN = cid * (tabA.shape[0] // 2)

    orow = orow0
    s_v = s_v0
    cnt_c = cnt_c0
    sw_c = sw_c0

    pltpu.sync_copy(wv.at[pl.ds(cid * 256, 128)], w0)
    pltpu.sync_copy(wv.at[pl.ds(cid * 256 + 128, 128)], w1)
    w0v = [w0[pl.ds(j * LANES, LANES)] for j in range(8)]
    w1v = [w1[pl.ds(j * LANES, LANES)] for j in range(8)]

    zv = jnp.zeros((LANES,), jnp.float32)

    def _zb(i, _):
        for j in range(128 // LANES):
            zbuf[i, pl.ds(j * LANES, LANES)] = zv
        return 0
    lax.fori_loop(0, ZROWS, _zb, 0)

    def _zr(i, _):
        stg_c[pl.ds(i * LANES, LANES)] = zv
        return 0
    lax.fori_loop(0, rpt // LANES, _zr, 0)
    for b in range(rpt // ZROWS):
        pltpu.sync_copy(zbuf, acc.at[pl.ds(tid * rpt + b * ZROWS, ZROWS)])
    pltpu.sync_copy(stg_c, cnt_acc.at[pl.ds(tid * rpt, rpt)])
    pltpu.sync_copy(stg_c, sw_acc.at[pl.ds(tid * rpt, rpt)])
    plsc.subcore_barrier()

    def _chunk(k, _):
        gc = (cid * E2 + tid * epw) // CHUNK + k
        ge = (tid * epw) // CHUNK + k
        pltpu.sync_copy(gAB.at[pl.ds(gc * 2 * CHUNK, 2 * CHUNK)], ab_v)
        pltpu.sync_copy(eaC.at[pl.ds(ge * 2 * CHUNK, 2 * CHUNK)], ea_v)
        cpA = pltpu.async_copy(tabA.at[ab_v.at[pl.ds(0, CHUNK)]], rA, semA)
        cpB = pltpu.async_copy(tabB.at[ab_v.at[pl.ds(CHUNK, CHUNK)]], rB,
                               semB)
        cpA.wait()
        cpB.wait()

        def _group(g, _):
            base = g * LANES
            a0v = plsc.bitcast(ea_v[pl.ds(base, LANES)], jnp.float32)
            a1v = plsc.bitcast(ea_v[pl.ds(CHUNK + base, LANES)], jnp.float32)
            gav = ab_v[pl.ds(base, LANES)]
            gbv = ab_v[pl.ds(CHUNK + base, LANES)]
            mv = jnp.where(gav == gbv, 0.0, 1.0)
            s_v[pl.ds(base, LANES)] = gav - cN
            cnt_c[pl.ds(base, LANES)] = mv
            sw_c[pl.ds(base, LANES)] = a0v * (1.0 - mv)
            for ii in range(LANES):
                i = base + ii
                a0 = a0v[ii]
                a1 = a1v[ii]
                m = mv[ii]
                for j in range(8):
                    sl = pl.ds(j * LANES, LANES)
                    v = a0 * w0v[j] + a1 * w1v[j] + rA[i, sl] + rB[i, sl]
                    orow[i, sl] = jnp.maximum(v, 0.0) * m
            return 0
        lax.fori_loop(0, CHUNK // LANES, _group, 0)
        pltpu.sync_copy(orow, acc.at[s_v], add=True)
        pltpu.sync_copy(cnt_c, cnt_acc.at[s_v], add=True)
        pltpu.sync_copy(sw_c, sw_acc.at[s_v], add=True)
        return 0
    lax.fori_loop(0, nchunk, _chunk, 0)
    plsc.subcore_barrier()

    pltpu.sync_copy(cnt_acc.at[pl.ds(tid * rpt, rpt)], stg_c)
    pltpu.sync_copy(stg_c, cnt_out.at[pl.ds(cid * NP + tid * rpt, rpt)])
    pltpu.sync_copy(sw_acc.at[pl.ds(tid * rpt, rpt)], stg_c)
    pltpu.sync_copy(stg_c, sw_out.at[pl.ds(cid * NP + tid * rpt, rpt)])
    for b in range(rpt // ZROWS):
        r = tid * rpt + b * ZROWS
        pltpu.sync_copy(acc.at[pl.ds(r, ZROWS)], zbuf)
        pltpu.sync_copy(zbuf, out.at[pl.ds(cid * NP + r, ZROWS)])


def _make_edge_call(NP):
    mesh = plsc.VectorSubcoreMesh(core_axis_name="c", subcore_axis_name="s")
    rpt = NP // NS
    scratch = (
        [pltpu.VMEM((2 * CHUNK,), jnp.int32)] * 2
        + [pltpu.VMEM((CHUNK,), jnp.int32)] * 2
        + [pltpu.VMEM((CHUNK, 128), jnp.float32)] * 4
        + [pltpu.VMEM((128,), jnp.float32)] * 2
        + [pltpu.VMEM((ZROWS, 128), jnp.float32)]
        + [pltpu.VMEM((CHUNK,), jnp.float32)] * 4
        + [pltpu.VMEM((rpt,), jnp.float32)]
        + [pltpu.VMEM_SHARED((NP, 128), jnp.float32)]
        + [pltpu.VMEM_SHARED((NP,), jnp.float32)] * 2
        + [pltpu.SemaphoreType.DMA] * 3
    )
    return functools.partial(
        pl.kernel, mesh=mesh,
        out_type=[
            jax.ShapeDtypeStruct((2 * NP, 128), jnp.float32),
            jax.ShapeDtypeStruct((2 * NP,), jnp.float32),
            jax.ShapeDtypeStruct((2 * NP,), jnp.float32),
        ],
        scratch_types=scratch,
        compiler_params=pltpu.CompilerParams(needs_layout_passes=False),
    )(_edge_kernel)


def _tables_body(H_ref, Wa_ref, ba_ref, Wb_ref, tabA_ref, tabB_ref):
    h = H_ref[...]
    for c in range(2):
        tabA_ref[c] = jnp.dot(h, Wa_ref[c],
                              preferred_element_type=jnp.float32) + ba_ref[c]
        tabB_ref[c] = jnp.dot(h, Wb_ref[c],
                              preferred_element_type=jnp.float32)


def _finalize_body(H_ref, S_ref, cl_ref, x_ref, y_ref,
                   W2to_ref, b2to_ref, W2fr_ref, b2fr_ref,
                   Wl_ref, wlc_ref, b1l_ref, W2l_ref, b2l_ref,
                   Wph_ref, Wpt_ref, Wpf_ref, Wpl_ref, Wpx_ref, b1p_ref,
                   W2p_ref, b2p_ref, Wd1_ref, bd1_ref, Wd2_ref, bd2_ref,
                   Hn_ref, F_ref, l6_ref, l2_ref):
    H = H_ref[...]
    dot = functools.partial(jnp.dot, preferred_element_type=jnp.float32)
    cnt_to = cl_ref[0, :, 0]
    cnt_fr = cl_ref[1, :, 0]
    lf = -cl_ref[0, :, 1]
    mess_to = (dot(S_ref[0], W2to_ref[...]) + cnt_to[:, None] * b2to_ref[...]) \
        / jnp.maximum(cnt_to, 1.0)[:, None]
    mess_fr = (dot(S_ref[1], W2fr_ref[...]) + cnt_fr[:, None] * b2fr_ref[...]) \
        / jnp.maximum(cnt_fr, 1.0)[:, None]
    loop = dot(jnp.maximum(dot(H, Wl_ref[...]) + lf[:, None] * wlc_ref[...]
                           + b1l_ref[...], 0.0), W2l_ref[...]) + b2l_ref[...]
    hid = jnp.maximum(dot(H, Wph_ref[...]) + dot(mess_to, Wpt_ref[...])
                      + dot(mess_fr, Wpf_ref[...]) + dot(loop, Wpl_ref[...])
                      + dot(x_ref[...], Wpx_ref[...]) + b1p_ref[...], 0.0)
    Hn = H + ALPHA * (dot(hid, W2p_ref[...]) + b2p_ref[...])
    F = dot(jnp.maximum(dot(Hn, Wd1_ref[...]) + bd1_ref[...], 0.0),
            Wd2_ref[...]) + bd2_ref[...]
    d = F - y_ref[...]
    d2 = d * d
    Hn_ref[...] = Hn
    F_ref[...] = F
    l6_ref[...] = jnp.full((1, 1, 128), jnp.sum(d2 * d2 * d2), jnp.float32)
    l2_ref[...] = jnp.full((1, 1, 128), jnp.sum(d2), jnp.float32)


def kernel(x, edge_index, edge_attr, y, epoch, n_epoch,
           phi_to_W1, phi_to_b1, phi_to_W2, phi_to_b2,
           phi_from_W1, phi_from_b1, phi_from_W2, phi_from_b2,
           loop_W1, loop_b1, loop_W2, loop_b2,
           psy_W1, psy_b1, psy_W2, psy_b2,
           dec_W1, dec_b1, dec_W2, dec_b2):
    N = x.shape[0]
    E = edge_index.shape[1]
    L = dec_W1.shape[1]
    K = dec_W1.shape[0]
    NP = ((N + NS * ZROWS - 1) // (NS * ZROWS)) * (NS * ZROWS)
    BN = 2000
    grid = N // BN

    src = edge_index[0]
    dst = edge_index[1]
    gA = jnp.stack([dst, src + N]).astype(jnp.int32).reshape(-1, CHUNK)
    gB = jnp.stack([src, dst + N]).astype(jnp.int32).reshape(-1, CHUNK)
    gAB = jnp.concatenate([gA, gB], axis=1).ravel()
    eaI = jax.lax.bitcast_convert_type(edge_attr.T, jnp.int32).reshape(-1, CHUNK)
    eaC = jnp.concatenate([eaI[:E // CHUNK], eaI[E // CHUNK:]], axis=1).ravel()
    edge_call = _make_edge_call(NP)

    tables_call = pl.pallas_call(
        _tables_body,
        grid=(grid,),
        in_specs=[
            pl.BlockSpec((BN, L), lambda b: (b, 0)),
            pl.BlockSpec((2, L, L), lambda b: (0, 0, 0)),
            pl.BlockSpec((2, 1, L), lambda b: (0, 0, 0)),
            pl.BlockSpec((2, L, L), lambda b: (0, 0, 0)),
        ],
        out_specs=[
            pl.BlockSpec((2, BN, L), lambda b: (0, b, 0)),
            pl.BlockSpec((2, BN, L), lambda b: (0, b, 0)),
        ],
        out_shape=[
            jax.ShapeDtypeStruct((2, N, L), jnp.float32),
            jax.ShapeDtypeStruct((2, N, L), jnp.float32),
        ],
    )

    wspec = pl.BlockSpec((L, L), lambda b: (0, 0))
    bspec = pl.BlockSpec((1, L), lambda b: (0, 0))
    fin_call = pl.pallas_call(
        _finalize_body,
        grid=(grid,),
        in_specs=[
            pl.BlockSpec((BN, L), lambda b: (b, 0)),
            pl.BlockSpec((2, BN, L), lambda b: (0, b, 0)),
            pl.BlockSpec((2, BN, 2), lambda b: (0, b, 0)),
            pl.BlockSpec((BN, 8), lambda b: (b, 0)),
            pl.BlockSpec((BN, 8), lambda b: (b, 0)),
            wspec, bspec, wspec, bspec,
            wspec, bspec, bspec, wspec, bspec,
            wspec, wspec, wspec, wspec,
            pl.BlockSpec((8, L), lambda b: (0, 0)), bspec,
            wspec, bspec, wspec, bspec,
            pl.BlockSpec((L, 8), lambda b: (0, 0)),
            pl.BlockSpec((1, 8), lambda b: (0, 0)),
        ],
        out_specs=[
            pl.BlockSpec((BN, L), lambda b: (b, 0)),
            pl.BlockSpec((BN, 8), lambda b: (b, 0)),
            pl.BlockSpec((1, 1, 128), lambda b: (b, 0, 0)),
            pl.BlockSpec((1, 1, 128), lambda b: (b, 0, 0)),
        ],
        out_shape=[
            jax.ShapeDtypeStruct((N, L), jnp.float32),
            jax.ShapeDtypeStruct((N, 8), jnp.float32),
            jax.ShapeDtypeStruct((grid, 1, 128), jnp.float32),
            jax.ShapeDtypeStruct((grid, 1, 128), jnp.float32),
        ],
    )

    x8 = jnp.pad(x, ((0, 0), (0, 8 - x.shape[1])))
    y8 = jnp.pad(y, ((0, 0), (0, 8 - y.shape[1])))

    H = jnp.zeros((N, L), jnp.float32)
    tl1 = jnp.float32(0.0)
    tl2 = jnp.float32(0.0)
    F = None
    for u in range(K):
        Wa = jnp.stack([phi_to_W1[u, :L], phi_from_W1[u, :L]])
        ba = jnp.stack([phi_to_b1[u], phi_from_b1[u]])[:, None, :]
        Wb = jnp.stack([phi_to_W1[u, L:2 * L], phi_from_W1[u, L:2 * L]])
        wv = jnp.stack([phi_to_W1[u, 2 * L:],
                        phi_from_W1[u, 2 * L:]]).ravel()
        tabA, tabB = tables_call(H, Wa, ba, Wb)
        acc, cnt_o, sw_o = edge_call(tabA.reshape(2 * N, L),
                                     tabB.reshape(2 * N, L), gAB, eaC, wv)
        S2 = acc.reshape(2, NP, L)[:, :N, :]
        cl = jnp.stack([cnt_o.reshape(2, NP)[:, :N],
                        sw_o.reshape(2, NP)[:, :N]], axis=-1)
        Wd2p = jnp.pad(dec_W2[u], ((0, 0), (0, 6)))
        bd2p = jnp.pad(dec_b2[u], ((0, 6)))[None, :]
        Wpxp = jnp.pad(psy_W1[u, 4 * L:], ((0, 5), (0, 0)))
        H, F8, l6, l2 = fin_call(
            H, S2, cl, x8, y8,
            phi_to_W2[u], phi_to_b2[u][None, :],
            phi_from_W2[u], phi_from_b2[u][None, :],
            loop_W1[u, :L] + loop_W1[u, L:2 * L], loop_W1[u, 2 * L][None, :],
            loop_b1[u][None, :], loop_W2[u], loop_b2[u][None, :],
            psy_W1[u, :L], psy_W1[u, L:2 * L], psy_W1[u, 2 * L:3 * L],
            psy_W1[u, 3 * L:4 * L], Wpxp, psy_b1[u][None, :],
            psy_W2[u], psy_b2[u][None, :],
            dec_W1[u], dec_b1[u][None, :], Wd2p, bd2p)
        F = F8[:, :2]
        w = GAMMA ** (K - u - 1)
        tl1 = tl1 + jnp.sum(l6[:, 0, 0]) ** (1.0 / 6.0) * w
        tl2 = tl2 + jnp.sqrt(jnp.sum(l2[:, 0, 0])) * w
    return (F, tl1, tl2)


# speedup vs baseline: 2.4559x; 1.0315x over previous
import functools
import jax
import jax.numpy as jnp
from jax import lax
from jax.experimental import pallas as pl
from jax.experimental.pallas import tpu as pltpu
from jax.experimental.pallas import tpu_sc as plsc

GAMMA = 0.5
ALPHA = 0.1

NC = 2
NS = 16
LANES = 16
CHUNK = 80
ZROWS = 16


def _edge_kernel(tabA, tabB, gAB, wv, out, cnt_out, sw_out,
                 ab_v, ea_v, s_v0, s_v1, rA, rB, orow0, orow1,
                 w0, w1, zbuf, cnt_c0, cnt_c1, sw_c0, sw_c1, stg_c,
                 acc, cnt_acc, sw_acc, semA, semB, semS):
    E2 = gAB.shape[0] // 8
    epw = E2 // NS
    nchunk = epw // CHUNK
    NP = acc.shape[0]
    rpt = NP // NS
    cid = lax.axis_index("c")
    tid = lax.axis_index("s")
    cN = cid * (tabA.shape[0] // 2)

    orow = orow0
    s_v = s_v0
    cnt_c = cnt_c0
    sw_c = sw_c0

    pltpu.sync_copy(wv.at[pl.ds(cid * 256, 128)], w0)
    pltpu.sync_copy(wv.at[pl.ds(cid * 256 + 128, 128)], w1)
    w0v = [w0[pl.ds(j * LANES, LANES)] for j in range(8)]
    w1v = [w1[pl.ds(j * LANES, LANES)] for j in range(8)]

    zv = jnp.zeros((LANES,), jnp.float32)

    def _zb(i, _):
        for j in range(128 // LANES):
            zbuf[i, pl.ds(j * LANES, LANES)] = zv
        return 0
    lax.fori_loop(0, ZROWS, _zb, 0)

    def _zr(i, _):
        stg_c[pl.ds(i * LANES, LANES)] = zv
        return 0
    lax.fori_loop(0, rpt // LANES, _zr, 0)
    for b in range(rpt // ZROWS):
        pltpu.sync_copy(zbuf, acc.at[pl.ds(tid * rpt + b * ZROWS, ZROWS)])
    pltpu.sync_copy(stg_c, cnt_acc.at[pl.ds(tid * rpt, rpt)])
    pltpu.sync_copy(stg_c, sw_acc.at[pl.ds(tid * rpt, rpt)])
    plsc.subcore_barrier()

    def _chunk(k, _):
        gc = (cid * E2 + tid * epw) // CHUNK + k
        pltpu.sync_copy(gAB.at[pl.ds(gc * 4 * CHUNK, 4 * CHUNK)], ab_v)
        cpA = pltpu.async_copy(tabA.at[ab_v.at[pl.ds(0, CHUNK)]], rA, semA)
        cpB = pltpu.async_copy(tabB.at[ab_v.at[pl.ds(CHUNK, CHUNK)]], rB,
                               semB)
        cpA.wait()
        cpB.wait()

        def _group(g, hs):
            base = g * LANES
            a0v = plsc.bitcast(ab_v[pl.ds(2 * CHUNK + base, LANES)],
                               jnp.float32)
            a1v = plsc.bitcast(ab_v[pl.ds(3 * CHUNK + base, LANES)],
                               jnp.float32)
            gav = ab_v[pl.ds(base, LANES)]
            gbv = ab_v[pl.ds(CHUNK + base, LANES)]
            mv = jnp.where(gav == gbv, 0.0, 1.0)
            s_v[pl.ds(base, LANES)] = gav - cN
            cnt_c[pl.ds(base, LANES)] = mv
            sw_c[pl.ds(base, LANES)] = a0v * (1.0 - mv)
            for ii in range(LANES):
                i = base + ii
                a0 = a0v[ii]
                a1 = a1v[ii]
                m = mv[ii]
                for j in range(8):
                    sl = pl.ds(j * LANES, LANES)
                    v = a0 * w0v[j] + a1 * w1v[j] + rA[i, sl] + rB[i, sl]
                    orow[i, sl] = jnp.maximum(v, 0.0) * m
            return jnp.minimum(hs, jnp.min(mv))
        hs = lax.fori_loop(0, CHUNK // LANES, _group, jnp.float32(1.0))
        pltpu.sync_copy(orow, acc.at[s_v], add=True)
        pltpu.sync_copy(cnt_c, cnt_acc.at[s_v], add=True)

        @pl.when(hs < 0.5)
        def _():
            pltpu.sync_copy(sw_c, sw_acc.at[s_v], add=True)
        return 0
    lax.fori_loop(0, nchunk, _chunk, 0)
    plsc.subcore_barrier()

    pltpu.sync_copy(cnt_acc.at[pl.ds(tid * rpt, rpt)], stg_c)
    pltpu.sync_copy(stg_c, cnt_out.at[pl.ds(cid * NP + tid * rpt, rpt)])
    pltpu.sync_copy(sw_acc.at[pl.ds(tid * rpt, rpt)], stg_c)
    pltpu.sync_copy(stg_c, sw_out.at[pl.ds(cid * NP + tid * rpt, rpt)])
    for b in range(rpt // ZROWS):
        r = tid * rpt + b * ZROWS
        pltpu.sync_copy(acc.at[pl.ds(r, ZROWS)], zbuf)
        pltpu.sync_copy(zbuf, out.at[pl.ds(cid * NP + r, ZROWS)])


def _make_edge_call(NP):
    mesh = plsc.VectorSubcoreMesh(core_axis_name="c", subcore_axis_name="s")
    rpt = NP // NS
    scratch = (
        [pltpu.VMEM((4 * CHUNK,), jnp.int32)]
        + [pltpu.VMEM((2 * CHUNK,), jnp.int32)]
        + [pltpu.VMEM((CHUNK,), jnp.int32)] * 2
        + [pltpu.VMEM((CHUNK, 128), jnp.float32)] * 4
        + [pltpu.VMEM((128,), jnp.float32)] * 2
        + [pltpu.VMEM((ZROWS, 128), jnp.float32)]
        + [pltpu.VMEM((CHUNK,), jnp.float32)] * 4
        + [pltpu.VMEM((rpt,), jnp.float32)]
        + [pltpu.VMEM_SHARED((NP, 128), jnp.float32)]
        + [pltpu.VMEM_SHARED((NP,), jnp.float32)] * 2
        + [pltpu.SemaphoreType.DMA] * 3
    )
    return functools.partial(
        pl.kernel, mesh=mesh,
        out_type=[
            jax.ShapeDtypeStruct((2 * NP, 128), jnp.float32),
            jax.ShapeDtypeStruct((2 * NP,), jnp.float32),
            jax.ShapeDtypeStruct((2 * NP,), jnp.float32),
        ],
        scratch_types=scratch,
        compiler_params=pltpu.CompilerParams(needs_layout_passes=False),
    )(_edge_kernel)


def _tables_body(H_ref, Wa_ref, ba_ref, Wb_ref, tabA_ref, tabB_ref):
    h = H_ref[...]
    for c in range(2):
        tabA_ref[c] = jnp.dot(h, Wa_ref[c],
                              preferred_element_type=jnp.float32) + ba_ref[c]
        tabB_ref[c] = jnp.dot(h, Wb_ref[c],
                              preferred_element_type=jnp.float32)


def _finalize_body(H_ref, S_ref, cl_ref, x_ref, y_ref,
                   W2to_ref, b2to_ref, W2fr_ref, b2fr_ref,
                   Wl_ref, wlc_ref, b1l_ref, W2l_ref, b2l_ref,
                   Wph_ref, Wpt_ref, Wpf_ref, Wpl_ref, Wpx_ref, b1p_ref,
                   W2p_ref, b2p_ref, Wd1_ref, bd1_ref, Wd2_ref, bd2_ref,
                   Hn_ref, F_ref, l6_ref, l2_ref):
    H = H_ref[...]
    dot = functools.partial(jnp.dot, preferred_element_type=jnp.float32)
    cnt_to = cl_ref[0, :, 0]
    cnt_fr = cl_ref[1, :, 0]
    lf = -cl_ref[0, :, 1]
    mess_to = (dot(S_ref[0], W2to_ref[...]) + cnt_to[:, None] * b2to_ref[...]) \
        / jnp.maximum(cnt_to, 1.0)[:, None]
    mess_fr = (dot(S_ref[1], W2fr_ref[...]) + cnt_fr[:, None] * b2fr_ref[...]) \
        / jnp.maximum(cnt_fr, 1.0)[:, None]
    loop = dot(jnp.maximum(dot(H, Wl_ref[...]) + lf[:, None] * wlc_ref[...]
                           + b1l_ref[...], 0.0), W2l_ref[...]) + b2l_ref[...]
    hid = jnp.maximum(dot(H, Wph_ref[...]) + dot(mess_to, Wpt_ref[...])
                      + dot(mess_fr, Wpf_ref[...]) + dot(loop, Wpl_ref[...])
                      + dot(x_ref[...], Wpx_ref[...]) + b1p_ref[...], 0.0)
    Hn = H + ALPHA * (dot(hid, W2p_ref[...]) + b2p_ref[...])
    F = dot(jnp.maximum(dot(Hn, Wd1_ref[...]) + bd1_ref[...], 0.0),
            Wd2_ref[...]) + bd2_ref[...]
    d = F - y_ref[...]
    d2 = d * d
    Hn_ref[...] = Hn
    F_ref[...] = F
    l6_ref[...] = jnp.full((1, 1, 128), jnp.sum(d2 * d2 * d2), jnp.float32)
    l2_ref[...] = jnp.full((1, 1, 128), jnp.sum(d2), jnp.float32)


def kernel(x, edge_index, edge_attr, y, epoch, n_epoch,
           phi_to_W1, phi_to_b1, phi_to_W2, phi_to_b2,
           phi_from_W1, phi_from_b1, phi_from_W2, phi_from_b2,
           loop_W1, loop_b1, loop_W2, loop_b2,
           psy_W1, psy_b1, psy_W2, psy_b2,
           dec_W1, dec_b1, dec_W2, dec_b2):
    N = x.shape[0]
    E = edge_index.shape[1]
    L = dec_W1.shape[1]
    K = dec_W1.shape[0]
    NP = ((N + NS * ZROWS - 1) // (NS * ZROWS)) * (NS * ZROWS)
    BN = 2000
    grid = N // BN

    src = edge_index[0]
    dst = edge_index[1]
    gAr = jnp.stack([dst, src + N]).astype(jnp.int32).reshape(2, -1, CHUNK)
    gBr = jnp.stack([src, dst + N]).astype(jnp.int32).reshape(2, -1, CHUNK)
    eaI = jax.lax.bitcast_convert_type(edge_attr.T,
                                       jnp.int32).reshape(2, -1, CHUNK)
    ea4 = jnp.broadcast_to(eaI.transpose(1, 0, 2)[None],
                           (2, E // CHUNK, 2, CHUNK))
    gAB = jnp.concatenate(
        [gAr[:, :, None], gBr[:, :, None], ea4], axis=2).ravel()
    edge_call = _make_edge_call(NP)

    tables_call = pl.pallas_call(
        _tables_body,
        grid=(grid,),
        in_specs=[
            pl.BlockSpec((BN, L), lambda b: (b, 0)),
            pl.BlockSpec((2, L, L), lambda b: (0, 0, 0)),
            pl.BlockSpec((2, 1, L), lambda b: (0, 0, 0)),
            pl.BlockSpec((2, L, L), lambda b: (0, 0, 0)),
        ],
        out_specs=[
            pl.BlockSpec((2, BN, L), lambda b: (0, b, 0)),
            pl.BlockSpec((2, BN, L), lambda b: (0, b, 0)),
        ],
        out_shape=[
            jax.ShapeDtypeStruct((2, N, L), jnp.float32),
            jax.ShapeDtypeStruct((2, N, L), jnp.float32),
        ],
    )

    wspec = pl.BlockSpec((L, L), lambda b: (0, 0))
    bspec = pl.BlockSpec((1, L), lambda b: (0, 0))
    fin_call = pl.pallas_call(
        _finalize_body,
        grid=(grid,),
        in_specs=[
            pl.BlockSpec((BN, L), lambda b: (b, 0)),
            pl.BlockSpec((2, BN, L), lambda b: (0, b, 0)),
            pl.BlockSpec((2, BN, 2), lambda b: (0, b, 0)),
            pl.BlockSpec((BN, 8), lambda b: (b, 0)),
            pl.BlockSpec((BN, 8), lambda b: (b, 0)),
            wspec, bspec, wspec, bspec,
            wspec, bspec, bspec, wspec, bspec,
            wspec, wspec, wspec, wspec,
            pl.BlockSpec((8, L), lambda b: (0, 0)), bspec,
            wspec, bspec, wspec, bspec,
            pl.BlockSpec((L, 8), lambda b: (0, 0)),
            pl.BlockSpec((1, 8), lambda b: (0, 0)),
        ],
        out_specs=[
            pl.BlockSpec((BN, L), lambda b: (b, 0)),
            pl.BlockSpec((BN, 8), lambda b: (b, 0)),
            pl.BlockSpec((1, 1, 128), lambda b: (b, 0, 0)),
            pl.BlockSpec((1, 1, 128), lambda b: (b, 0, 0)),
        ],
        out_shape=[
            jax.ShapeDtypeStruct((N, L), jnp.float32),
            jax.ShapeDtypeStruct((N, 8), jnp.float32),
            jax.ShapeDtypeStruct((grid, 1, 128), jnp.float32),
            jax.ShapeDtypeStruct((grid, 1, 128), jnp.float32),
        ],
    )

    x8 = jnp.pad(x, ((0, 0), (0, 8 - x.shape[1])))
    y8 = jnp.pad(y, ((0, 0), (0, 8 - y.shape[1])))

    H = jnp.zeros((N, L), jnp.float32)
    tl1 = jnp.float32(0.0)
    tl2 = jnp.float32(0.0)
    F = None
    for u in range(K):
        Wa = jnp.stack([phi_to_W1[u, :L], phi_from_W1[u, :L]])
        ba = jnp.stack([phi_to_b1[u], phi_from_b1[u]])[:, None, :]
        Wb = jnp.stack([phi_to_W1[u, L:2 * L], phi_from_W1[u, L:2 * L]])
        wv = jnp.stack([phi_to_W1[u, 2 * L:],
                        phi_from_W1[u, 2 * L:]]).ravel()
        tabA, tabB = tables_call(H, Wa, ba, Wb)
        acc, cnt_o, sw_o = edge_call(tabA.reshape(2 * N, L),
                                     tabB.reshape(2 * N, L), gAB, wv)
        S2 = acc.reshape(2, NP, L)[:, :N, :]
        cl = jnp.stack([cnt_o.reshape(2, NP)[:, :N],
                        sw_o.reshape(2, NP)[:, :N]], axis=-1)
        Wd2p = jnp.pad(dec_W2[u], ((0, 0), (0, 6)))
        bd2p = jnp.pad(dec_b2[u], ((0, 6)))[None, :]
        Wpxp = jnp.pad(psy_W1[u, 4 * L:], ((0, 5), (0, 0)))
        H, F8, l6, l2 = fin_call(
            H, S2, cl, x8, y8,
            phi_to_W2[u], phi_to_b2[u][None, :],
            phi_from_W2[u], phi_from_b2[u][None, :],
            loop_W1[u, :L] + loop_W1[u, L:2 * L], loop_W1[u, 2 * L][None, :],
            loop_b1[u][None, :], loop_W2[u], loop_b2[u][None, :],
            psy_W1[u, :L], psy_W1[u, L:2 * L], psy_W1[u, 2 * L:3 * L],
            psy_W1[u, 3 * L:4 * L], Wpxp, psy_b1[u][None, :],
            psy_W2[u], psy_b2[u][None, :],
            dec_W1[u], dec_b1[u][None, :], Wd2p, bd2p)
        F = F8[:, :2]
        w = GAMMA ** (K - u - 1)
        tl1 = tl1 + jnp.sum(l6[:, 0, 0]) ** (1.0 / 6.0) * w
        tl2 = tl2 + jnp.sqrt(jnp.sum(l2[:, 0, 0])) * w
    return (F, tl1, tl2)
